# baseline (device time: 24124 ns/iter reference)
import jax
import jax.numpy as jnp
from jax import lax
from jax.experimental import pallas as pl
from jax.experimental.pallas import tpu as pltpu

N_DEV = 4


def kernel(x, router_W, route_idx, expert_W, shared_W):
    n, d = x.shape
    e_per, _, h = expert_W.shape
    n_exp = router_W.shape[1]
    m_per = n // N_DEV

    def body(x_ref, rw_ref, idx_ref, ew_ref, sw_ref, out_ref,
             p_ref, send_ref, recv_ref, send_sems, recv_sems):
        my = lax.axis_index("i")
        left = (my + N_DEV - 1) % N_DEV
        right = (my + 1) % N_DEV

        barrier_sem = pltpu.get_barrier_semaphore()
        for nbr in (left, right):
            pl.semaphore_signal(barrier_sem, inc=1, device_id=(nbr,),
                                device_id_type=pl.DeviceIdType.MESH)
        pl.semaphore_wait(barrier_sem, 2)

        xb = x_ref[:, :].astype(jnp.bfloat16)

        scores = jnp.dot(xb, rw_ref[:, :].astype(jnp.bfloat16),
                         preferred_element_type=jnp.float32)
        smax = jnp.max(scores, axis=-1, keepdims=True)
        ex = jnp.exp(scores - smax)
        probs = ex / jnp.sum(ex, axis=-1, keepdims=True)

        idx = idx_ref[:, :]
        eidx = lax.broadcasted_iota(jnp.int32, (n, n_exp), 1)
        gate = jnp.sum(jnp.where(eidx == idx, probs, 0.0),
                       axis=-1, keepdims=True)

        partial = jnp.zeros((n, h), jnp.float32)
        for el in range(e_per):
            ge = my * e_per + el
            y = jnp.dot(xb, ew_ref[el].astype(jnp.bfloat16),
                        preferred_element_type=jnp.float32)
            partial = partial + y * jnp.where(idx == ge, gate, 0.0)
        p_ref[:, :] = partial

        for t in range(N_DEV - 1):
            c_send = (my - t - 1) % N_DEV
            chunk = p_ref[pl.ds(c_send * m_per, m_per), :]
            if t == 0:
                send_ref[:, :] = chunk
            else:
                send_ref[:, :] = recv_ref[t - 1] + chunk
            rdma = pltpu.make_async_remote_copy(
                src_ref=send_ref,
                dst_ref=recv_ref.at[t],
                send_sem=send_sems.at[t],
                recv_sem=recv_sems.at[t],
                device_id=(right,),
                device_id_type=pl.DeviceIdType.MESH,
            )
            rdma.start()
            rdma.wait()

        xr = x_ref[pl.ds(my * m_per, m_per), :].astype(jnp.bfloat16)
        shared = jnp.dot(xr, sw_ref[:, :].astype(jnp.bfloat16),
                         preferred_element_type=jnp.float32)
        own = p_ref[pl.ds(my * m_per, m_per), :]
        out_ref[:, :] = recv_ref[N_DEV - 2] + own + shared

    return pl.pallas_call(
        body,
        out_shape=jax.ShapeDtypeStruct((m_per, h), jnp.float32),
        in_specs=[pl.BlockSpec(memory_space=pltpu.VMEM)] * 5,
        out_specs=pl.BlockSpec(memory_space=pltpu.VMEM),
        scratch_shapes=[
            pltpu.VMEM((n, h), jnp.float32),
            pltpu.VMEM((m_per, h), jnp.float32),
            pltpu.VMEM((N_DEV - 1, m_per, h), jnp.float32),
            pltpu.SemaphoreType.DMA((N_DEV - 1,)),
            pltpu.SemaphoreType.DMA((N_DEV - 1,)),
        ],
        compiler_params=pltpu.CompilerParams(collective_id=0),
    )(x, router_W, route_idx, expert_W, shared_W)


# device time: 15555 ns/iter; 1.5509x vs baseline; 1.5509x over previous
import jax
import jax.numpy as jnp
from jax import lax
from jax.experimental import pallas as pl
from jax.experimental.pallas import tpu as pltpu

N_DEV = 4


def kernel(x, router_W, route_idx, expert_W, shared_W):
    n, d = x.shape
    e_per, _, h = expert_W.shape
    n_exp = router_W.shape[1]
    m_per = n // N_DEV

    def body(x_ref, rw_ref, idx_ref, ew_ref, sw_ref, out_ref,
             p_ref, send_ref, recv_ref, send_sems, recv_sems):
        my = lax.axis_index("i")

        barrier_sem = pltpu.get_barrier_semaphore()
        for k in range(1, N_DEV):
            pl.semaphore_signal(barrier_sem, inc=1,
                                device_id=((my + k) % N_DEV,),
                                device_id_type=pl.DeviceIdType.MESH)
        pl.semaphore_wait(barrier_sem, N_DEV - 1)

        xb = x_ref[:, :].astype(jnp.bfloat16)

        scores = jnp.dot(xb, rw_ref[:, :].astype(jnp.bfloat16),
                         preferred_element_type=jnp.float32)
        smax = jnp.max(scores, axis=-1, keepdims=True)
        ex = jnp.exp(scores - smax)
        probs = ex / jnp.sum(ex, axis=-1, keepdims=True)

        idx = idx_ref[:, :]
        eidx = lax.broadcasted_iota(jnp.int32, (n, n_exp), 1)
        gate = jnp.sum(jnp.where(eidx == idx, probs, 0.0),
                       axis=-1, keepdims=True)

        xw = jnp.concatenate(
            [(jnp.where(idx == my * e_per + el, gate, 0.0) * x_ref[:, :]
              ).astype(jnp.bfloat16) for el in range(e_per)],
            axis=1)
        ew = ew_ref[:, :, :].astype(jnp.bfloat16).reshape(e_per * d, h)
        partial = jnp.dot(xw, ew, preferred_element_type=jnp.float32)
        p_ref[:, :] = partial.astype(jnp.bfloat16)

        rdmas = []
        for k in range(1, N_DEV):
            t = (my + k) % N_DEV
            send_ref[k - 1, :, :] = p_ref[pl.ds(t * m_per, m_per), :]
            rdma = pltpu.make_async_remote_copy(
                src_ref=send_ref.at[k - 1],
                dst_ref=recv_ref.at[N_DEV - 1 - k],
                send_sem=send_sems.at[k - 1],
                recv_sem=recv_sems.at[N_DEV - 1 - k],
                device_id=(t,),
                device_id_type=pl.DeviceIdType.MESH,
            )
            rdma.start()
            rdmas.append(rdma)

        xr = x_ref[pl.ds(my * m_per, m_per), :].astype(jnp.bfloat16)
        shared = jnp.dot(xr, sw_ref[:, :].astype(jnp.bfloat16),
                         preferred_element_type=jnp.float32)
        own = p_ref[pl.ds(my * m_per, m_per), :].astype(jnp.float32)

        for rdma in rdmas:
            rdma.wait_recv()
        acc = own + shared
        for s in range(N_DEV - 1):
            acc = acc + recv_ref[s].astype(jnp.float32)
        out_ref[:, :] = acc

        for rdma in rdmas:
            rdma.wait_send()

    return pl.pallas_call(
        body,
        out_shape=jax.ShapeDtypeStruct((m_per, h), jnp.float32),
        in_specs=[pl.BlockSpec(memory_space=pltpu.VMEM)] * 5,
        out_specs=pl.BlockSpec(memory_space=pltpu.VMEM),
        scratch_shapes=[
            pltpu.VMEM((n, h), jnp.bfloat16),
            pltpu.VMEM((N_DEV - 1, m_per, h), jnp.bfloat16),
            pltpu.VMEM((N_DEV - 1, m_per, h), jnp.bfloat16),
            pltpu.SemaphoreType.DMA((N_DEV - 1,)),
            pltpu.SemaphoreType.DMA((N_DEV - 1,)),
        ],
        compiler_params=pltpu.CompilerParams(collective_id=0),
    )(x, router_W, route_idx, expert_W, shared_W)
